# baseline (speedup 1.0000x reference)
import functools

import jax
import jax.numpy as jnp
from jax import lax
from jax.experimental import pallas as pl
from jax.experimental.pallas import tpu as pltpu
from jax.experimental.pallas import tpu_sc as plsc

BBLK = 256
NC, NS, LANES = 2, 16, 16
NW = NC * NS
CH = 8


def _sims_argmax_body(z_ref, cb_ref, idx_ref):
    bblk, f_total, d = z_ref.shape
    _, k_total, _ = cb_ref.shape
    for f in range(f_total):
        zf = z_ref[:, f, :]
        sims = lax.dot_general(
            zf, cb_ref[f], (((1,), (1,)), ((), ())),
            preferred_element_type=jnp.float32,
        )
        am = jnp.argmax(jnp.abs(sims), axis=1).astype(jnp.int32)
        idx_ref[f, :] = am + f * k_total


def _tc_sims_argmax(z, codebooks):
    b, f, d = z.shape
    return pl.pallas_call(
        _sims_argmax_body,
        grid=(b // BBLK,),
        in_specs=[
            pl.BlockSpec((BBLK, f, d), lambda i: (i, 0, 0)),
            pl.BlockSpec(codebooks.shape, lambda i: (0, 0, 0)),
        ],
        out_specs=pl.BlockSpec((f, BBLK), lambda i: (0, i)),
        out_shape=jax.ShapeDtypeStruct((f, b), jnp.int32),
        compiler_params=pltpu.CompilerParams(
            dimension_semantics=("arbitrary",),
        ),
    )(z, codebooks)


def _make_sc_gather(b, d, f_total):
    rows_per_w = b // NW
    n_ch = rows_per_w // CH
    mesh = plsc.VectorSubcoreMesh(
        core_axis_name="c", subcore_axis_name="s",
        num_cores=NC, num_subcores=NS,
    )

    @functools.partial(
        pl.kernel,
        out_type=jax.ShapeDtypeStruct((b, d), jnp.float32),
        mesh=mesh,
        scratch_types=[
            pltpu.VMEM((f_total, rows_per_w), jnp.int32),
            pltpu.VMEM((f_total, CH, d), jnp.float32),
            pltpu.SemaphoreType.DMA,
            pltpu.SemaphoreType.DMA,
        ],
    )
    def sc_gather(tbl_hbm, idx_hbm, out_hbm, idx_v, w_v, gsem, wsem):
        wid = lax.axis_index("s") * NC + lax.axis_index("c")
        base = wid * rows_per_w
        pltpu.sync_copy(idx_hbm.at[:, pl.ds(base, rows_per_w)], idx_v)

        def gathers(c):
            return [
                pltpu.async_copy(
                    tbl_hbm.at[idx_v.at[ff, pl.ds(c * CH, CH)]],
                    w_v.at[ff],
                    gsem,
                )
                for ff in range(f_total)
            ]

        def writeback(c):
            cp = pltpu.make_async_copy(
                w_v.at[0],
                out_hbm.at[pl.ds(base + c * CH, CH), :],
                wsem,
            )
            cp.start()
            return cp

        def multibind():
            def col_fix(j, carry):
                for r in range(CH):
                    sl = (r, pl.ds(j * LANES, LANES))
                    x = w_v[(0,) + sl] * w_v[(1,) + sl]
                    y = w_v[(2,) + sl] * w_v[(3,) + sl]
                    w_v[(0,) + sl] = x * y
                return carry

            lax.fori_loop(0, d // LANES, col_fix, 0)

        prev_wb = None
        for c in range(n_ch):
            if prev_wb is not None:
                prev_wb.wait()
            cps = gathers(c)
            for cp in cps:
                cp.wait()
            multibind()
            prev_wb = writeback(c)
        prev_wb.wait()

    return sc_gather


@jax.jit
def kernel(z, codebooks):
    b, f, d = z.shape
    k = codebooks.shape[1]
    fidx = _tc_sims_argmax(z, codebooks)
    tbl = codebooks.reshape(f * k, d)
    return _make_sc_gather(b, d, f)(tbl, fidx)


# speedup vs baseline: 1.3876x; 1.3876x over previous
import functools

import jax
import jax.numpy as jnp
from jax import lax
from jax.experimental import pallas as pl
from jax.experimental.pallas import tpu as pltpu
from jax.experimental.pallas import tpu_sc as plsc

BBLK = 256
NC, NS, LANES = 2, 16, 16
NW = NC * NS
CH = 4


def _sims_argmax_body(z_ref, cb_ref, idx_ref):
    bblk, f_total, d = z_ref.shape
    _, k_total, _ = cb_ref.shape
    for f in range(f_total):
        zf = z_ref[:, f, :]
        sims = lax.dot_general(
            zf, cb_ref[f], (((1,), (1,)), ((), ())),
            preferred_element_type=jnp.float32,
        )
        am = jnp.argmax(jnp.abs(sims), axis=1).astype(jnp.int32)
        idx_ref[f, :] = am + f * k_total


def _tc_sims_argmax(z, codebooks):
    b, f, d = z.shape
    return pl.pallas_call(
        _sims_argmax_body,
        grid=(b // BBLK,),
        in_specs=[
            pl.BlockSpec((BBLK, f, d), lambda i: (i, 0, 0)),
            pl.BlockSpec(codebooks.shape, lambda i: (0, 0, 0)),
        ],
        out_specs=pl.BlockSpec((f, BBLK), lambda i: (0, i)),
        out_shape=jax.ShapeDtypeStruct((f, b), jnp.int32),
        compiler_params=pltpu.CompilerParams(
            dimension_semantics=("arbitrary",),
        ),
    )(z, codebooks)


def _make_sc_gather(b, d, f_total):
    rows_per_w = b // NW
    n_ch = rows_per_w // CH
    mesh = plsc.VectorSubcoreMesh(
        core_axis_name="c", subcore_axis_name="s",
        num_cores=NC, num_subcores=NS,
    )

    @functools.partial(
        pl.kernel,
        out_type=jax.ShapeDtypeStruct((b, d), jnp.float32),
        mesh=mesh,
        scratch_types=[
            pltpu.VMEM((f_total, rows_per_w), jnp.int32),
            pltpu.VMEM((2, f_total, CH, d), jnp.float32),
            pltpu.SemaphoreType.DMA((2,)),
            pltpu.SemaphoreType.DMA((2,)),
        ],
    )
    def sc_gather(tbl_hbm, idx_hbm, out_hbm, idx_v, w_v, gsem, wsem):
        wid = lax.axis_index("s") * NC + lax.axis_index("c")
        base = wid * rows_per_w
        pltpu.sync_copy(idx_hbm.at[:, pl.ds(base, rows_per_w)], idx_v)

        def gathers(c, slot):
            return [
                pltpu.async_copy(
                    tbl_hbm.at[idx_v.at[ff, pl.ds(c * CH, CH)]],
                    w_v.at[slot, ff],
                    gsem.at[slot],
                )
                for ff in range(f_total)
            ]

        def writeback(c, slot):
            cp = pltpu.make_async_copy(
                w_v.at[slot, 0],
                out_hbm.at[pl.ds(base + c * CH, CH), :],
                wsem.at[slot],
            )
            cp.start()
            return cp

        def multibind(slot):
            def col_fix(j, carry):
                for r in range(CH):
                    sl = (r, pl.ds(j * LANES, LANES))
                    x = w_v[(slot, 0) + sl] * w_v[(slot, 1) + sl]
                    y = w_v[(slot, 2) + sl] * w_v[(slot, 3) + sl]
                    w_v[(slot, 0) + sl] = x * y
                return carry

            lax.fori_loop(0, d // LANES, col_fix, 0)

        pending_g = {0: gathers(0, 0)}
        pending_wb = {}
        for c in range(n_ch):
            slot = c % 2
            if c + 1 < n_ch:
                if c - 1 in pending_wb:
                    pending_wb.pop(c - 1).wait()
                pending_g[c + 1] = gathers(c + 1, 1 - slot)
            for cp in pending_g.pop(c):
                cp.wait()
            multibind(slot)
            pending_wb[c] = writeback(c, slot)
        for c in sorted(pending_wb):
            pending_wb.pop(c).wait()

    return sc_gather


@jax.jit
def kernel(z, codebooks):
    b, f, d = z.shape
    k = codebooks.shape[1]
    fidx = _tc_sims_argmax(z, codebooks)
    tbl = codebooks.reshape(f * k, d)
    return _make_sc_gather(b, d, f)(tbl, fidx)


# speedup vs baseline: 2.2704x; 1.6363x over previous
import functools

import jax
import jax.numpy as jnp
from jax import lax
from jax.experimental import pallas as pl
from jax.experimental.pallas import tpu as pltpu

BBLK = 512


def _cleanup_body(z_ref, cb_ref, cbh_ref, out_ref):
    bblk, f_total, d = z_ref.shape
    _, k_total, _ = cb_ref.shape
    acc = None
    for f in range(f_total):
        zf = z_ref[:, f, :]
        sims = lax.dot_general(
            zf, cb_ref[f], (((1,), (1,)), ((), ())),
            preferred_element_type=jnp.float32,
        )
        idx = jnp.argmax(jnp.abs(sims), axis=1)
        onehot = (
            idx[:, None] == lax.broadcasted_iota(jnp.int32, (bblk, k_total), 1)
        ).astype(jnp.bfloat16)
        wf = lax.dot_general(
            onehot, cbh_ref[f], (((1,), (0,)), ((), ())),
            preferred_element_type=jnp.float32,
        )
        acc = wf if acc is None else acc * wf
    out_ref[...] = acc


@jax.jit
def kernel(z, codebooks):
    b, f, d = z.shape
    return pl.pallas_call(
        _cleanup_body,
        grid=(b // BBLK,),
        in_specs=[
            pl.BlockSpec((BBLK, f, d), lambda i: (i, 0, 0)),
            pl.BlockSpec(codebooks.shape, lambda i: (0, 0, 0)),
            pl.BlockSpec(codebooks.shape, lambda i: (0, 0, 0)),
        ],
        out_specs=pl.BlockSpec((BBLK, d), lambda i: (i, 0)),
        out_shape=jax.ShapeDtypeStruct((b, d), jnp.float32),
        compiler_params=pltpu.CompilerParams(
            dimension_semantics=("arbitrary",),
        ),
    )(z, codebooks, codebooks.astype(jnp.bfloat16))
